# baseline (speedup 1.0000x reference)
import functools

import jax
import jax.numpy as jnp
from jax import lax
from jax.experimental import pallas as pl
from jax.experimental.pallas import tpu as pltpu
from jax.experimental.pallas import tpu_sc as plsc

B = 16384
P = 12
V = 28
D = 24
DS = 25
NTOK = B * P
NW = 32
TOK_W = NTOK // NW
CT = 1536
NCH = TOK_W // CT
GRP = CT // 16
LANES = 16


def _build_fused_body(word_ref, pos_ref, out_ref):
    s = pos_ref[...] + word_ref[...][None, :, :]
    out_ref[...] = jnp.pad(s, ((0, 0), (0, 0), (0, DS - D)))


_build_fused = pl.pallas_call(
    _build_fused_body,
    out_shape=jax.ShapeDtypeStruct((P, V, DS), jnp.float32),
)

_sc_mesh = plsc.VectorSubcoreMesh(core_axis_name="c", subcore_axis_name="s")


@functools.partial(
    pl.kernel,
    mesh=_sc_mesh,
    compiler_params=pltpu.CompilerParams(needs_layout_passes=False),
    out_type=jax.ShapeDtypeStruct((D, NTOK), jnp.float32),
    scratch_types=[
        pltpu.VMEM((P * V * DS,), jnp.float32),
        pltpu.VMEM((TOK_W,), jnp.int32),
        pltpu.VMEM((D, CT), jnp.float32),
        pltpu.VMEM((D, CT), jnp.float32),
        pltpu.SemaphoreType.DMA,
        pltpu.SemaphoreType.DMA,
    ],
)
def _sc_embed(fused_hbm, x_hbm, out_hbm, fused_v, x_v, buf0, buf1, sem0, sem1):
    wid = lax.axis_index("s") * 2 + lax.axis_index("c")
    base = pl.multiple_of(wid * TOK_W, TOK_W)
    pltpu.sync_copy(fused_hbm, fused_v)
    pltpu.sync_copy(x_hbm.at[pl.ds(base, TOK_W)], x_v)

    lane = lax.iota(jnp.int32, LANES)

    bufs = (buf0, buf1)
    sems = (sem0, sem1)

    def compute_chunk(c, buf):
        def group(g, carry):
            t = pl.multiple_of(c * CT + g * LANES, LANES)
            xv = x_v[pl.ds(t, LANES)]
            lv = lax.rem(t + lane, P)
            row_s = (lv * V + xv) * DS
            o = pl.multiple_of(g * LANES, LANES)
            for dd in range(D):
                buf[dd, pl.ds(o, LANES)] = plsc.load_gather(
                    fused_v, [row_s + dd])
            return carry

        lax.fori_loop(0, GRP, group, 0)

    copies = []
    for c in range(NCH):
        bsel = c % 2
        if c >= 2:
            copies[c - 2].wait()
        compute_chunk(c, bufs[bsel])
        off = pl.multiple_of(base + c * CT, CT)
        copies.append(
            pltpu.async_copy(bufs[bsel], out_hbm.at[:, pl.ds(off, CT)],
                             sems[bsel]))
    copies[-2].wait()
    copies[-1].wait()


def kernel(X, word_table, pos_table):
    fused = _build_fused(word_table, pos_table[:, None, :])
    fused_flat = fused.reshape(P * V * DS)
    x_flat = X.reshape(NTOK).astype(jnp.int32)
    out_t = _sc_embed(fused_flat, x_flat)
    return out_t.reshape(D, B, P).transpose(1, 2, 0)


# speedup vs baseline: 1.2086x; 1.2086x over previous
import functools

import jax
import jax.numpy as jnp
from jax import lax
from jax.experimental import pallas as pl
from jax.experimental.pallas import tpu as pltpu
from jax.experimental.pallas import tpu_sc as plsc

B = 16384
P = 12
V = 28
D = 24
DS = 25
NTOK = B * P
NW = 32
TOK_W = NTOK // NW
CT = 1536
NCH = TOK_W // CT
GRP = CT // 16
LANES = 16


def _build_fused_body(word_ref, pos_ref, out_ref):
    s = pos_ref[...] + word_ref[...][None, :, :]
    out_ref[...] = jnp.pad(s, ((0, 0), (0, 0), (0, DS - D)))


_build_fused = pl.pallas_call(
    _build_fused_body,
    out_shape=jax.ShapeDtypeStruct((P, V, DS), jnp.float32),
)

_sc_mesh = plsc.VectorSubcoreMesh(core_axis_name="c", subcore_axis_name="s")


@functools.partial(
    pl.kernel,
    mesh=_sc_mesh,
    compiler_params=pltpu.CompilerParams(needs_layout_passes=False),
    out_type=jax.ShapeDtypeStruct((NTOK * DS,), jnp.float32),
    scratch_types=[
        pltpu.VMEM((P * V * DS,), jnp.float32),
        pltpu.VMEM((TOK_W,), jnp.int32),
        pltpu.VMEM((CT * DS,), jnp.float32),
        pltpu.VMEM((CT * DS,), jnp.float32),
        pltpu.SemaphoreType.DMA,
        pltpu.SemaphoreType.DMA,
    ],
)
def _sc_embed(fused_hbm, x_hbm, out_hbm, fused_v, x_v, buf0, buf1, sem0, sem1):
    wid = lax.axis_index("s") * 2 + lax.axis_index("c")
    base = pl.multiple_of(wid * TOK_W, TOK_W)
    pltpu.sync_copy(fused_hbm, fused_v)
    pltpu.sync_copy(x_hbm.at[pl.ds(base, TOK_W)], x_v)

    lane = lax.iota(jnp.int32, LANES)
    lane_s = lane * DS

    bufs = (buf0, buf1)
    sems = (sem0, sem1)

    def compute_chunk(c, buf):
        def group(g, carry):
            t = pl.multiple_of(c * CT + g * LANES, LANES)
            xv = x_v[pl.ds(t, LANES)]
            lv = lax.rem(t + lane, P)
            row_s = (lv * V + xv) * DS
            st = lane_s + g * (LANES * DS)
            for dd in range(D):
                vals = plsc.load_gather(fused_v, [row_s + dd])
                plsc.store_scatter(buf, [st + dd], vals)
            return carry

        lax.fori_loop(0, GRP, group, 0)

    copies = []
    for c in range(NCH):
        bsel = c % 2
        if c >= 2:
            copies[c - 2].wait()
        compute_chunk(c, bufs[bsel])
        off = pl.multiple_of((base + c * CT) * DS, CT * DS)
        copies.append(
            pltpu.async_copy(bufs[bsel], out_hbm.at[pl.ds(off, CT * DS)],
                             sems[bsel]))
    copies[-2].wait()
    copies[-1].wait()


def kernel(X, word_table, pos_table):
    fused = _build_fused(word_table, pos_table[:, None, :])
    fused_flat = fused.reshape(P * V * DS)
    x_flat = X.reshape(NTOK).astype(jnp.int32)
    out_p = _sc_embed(fused_flat, x_flat)
    return out_p.reshape(NTOK, DS)[:, :D].reshape(B, P, D)


# speedup vs baseline: 1.9971x; 1.6523x over previous
import functools

import jax
import jax.numpy as jnp
from jax import lax
from jax.experimental import pallas as pl
from jax.experimental.pallas import tpu as pltpu
from jax.experimental.pallas import tpu_sc as plsc

B = 16384
P = 12
V = 28
D = 24
DS = 25
FW = P * V * DS
NTOK = B * P
NW = 32
TOK_W = NTOK // NW
CT = 1536
NCH = TOK_W // CT
GRP = CT // 16
LANES = 16

_sc_mesh = plsc.VectorSubcoreMesh(core_axis_name="c", subcore_axis_name="s")


@functools.partial(
    pl.kernel,
    mesh=_sc_mesh,
    compiler_params=pltpu.CompilerParams(needs_layout_passes=False),
    out_type=jax.ShapeDtypeStruct((NTOK * D,), jnp.float32),
    scratch_types=[
        pltpu.VMEM((V * D + P * D,), jnp.float32),
        pltpu.VMEM((FW,), jnp.float32),
        pltpu.VMEM((TOK_W,), jnp.int32),
        pltpu.VMEM((CT * D,), jnp.float32),
        pltpu.VMEM((CT * D,), jnp.float32),
        pltpu.SemaphoreType.DMA,
        pltpu.SemaphoreType.DMA,
    ],
)
def _sc_embed(wp_hbm, x_hbm, out_hbm, wp_v, fused_v, x_v, buf0, buf1,
              sem0, sem1):
    wid = lax.axis_index("s") * 2 + lax.axis_index("c")
    base = pl.multiple_of(wid * TOK_W, TOK_W)
    pltpu.sync_copy(wp_hbm, wp_v)
    pltpu.sync_copy(x_hbm.at[pl.ds(base, TOK_W)], x_v)

    lane = lax.iota(jnp.int32, LANES)

    def build(i, carry):
        p = i * LANES + lane
        r = p // DS
        c = jnp.minimum(p - r * DS, D - 1)
        l = r // V
        v = r - l * V
        wv = plsc.load_gather(wp_v, [v * D + c])
        pv = plsc.load_gather(wp_v, [V * D + l * D + c])
        fused_v[pl.ds(i * LANES, LANES)] = wv + pv
        return carry

    lax.fori_loop(0, FW // LANES, build, 0)

    half = lane // 8
    c1 = jnp.where(lane < 8, lane + 16, lane - 8)
    c2 = lane + 8

    bufs = (buf0, buf1)
    sems = (sem0, sem1)

    def compute_chunk(c, buf):
        def group(g, carry):
            t = pl.multiple_of(c * CT + g * LANES, LANES)
            xv = x_v[pl.ds(t, LANES)]
            lv = lax.rem(t + lane, P)
            row = (lv * V + xv) * DS
            ob = g * (LANES * D)
            for q in range(8):
                a = jnp.broadcast_to(row[2 * q], (LANES,))
                b = jnp.broadcast_to(row[2 * q + 1], (LANES,))
                ab = jnp.where(half == 0, a, b)
                o = ob + q * 48
                buf[pl.ds(o, LANES)] = plsc.load_gather(fused_v, [a + lane])
                buf[pl.ds(o + 16, LANES)] = plsc.load_gather(fused_v, [ab + c1])
                buf[pl.ds(o + 32, LANES)] = plsc.load_gather(fused_v, [b + c2])
            return carry

        lax.fori_loop(0, GRP, group, 0)

    copies = []
    for c in range(NCH):
        bsel = c % 2
        if c >= 2:
            copies[c - 2].wait()
        compute_chunk(c, bufs[bsel])
        off = pl.multiple_of((base + c * CT) * D, CT * D)
        copies.append(
            pltpu.async_copy(bufs[bsel], out_hbm.at[pl.ds(off, CT * D)],
                             sems[bsel]))
    copies[-2].wait()
    copies[-1].wait()


def kernel(X, word_table, pos_table):
    wp = jnp.concatenate([word_table.reshape(V * D), pos_table.reshape(P * D)])
    x_flat = X.reshape(NTOK).astype(jnp.int32)
    out_flat = _sc_embed(wp, x_flat)
    return out_flat.reshape(B, P, D)


# speedup vs baseline: 2.2369x; 1.1201x over previous
import functools

import jax
import jax.numpy as jnp
from jax import lax
from jax.experimental import pallas as pl
from jax.experimental.pallas import tpu as pltpu
from jax.experimental.pallas import tpu_sc as plsc

B = 16384
P = 12
V = 28
D = 24
DS = 25
FW = P * V * DS
NTOK = B * P
NW = 32
TOK_W = NTOK // NW
CT = 1536
NCH = TOK_W // CT
GRP = CT // 16
LANES = 16

_sc_mesh = plsc.VectorSubcoreMesh(core_axis_name="c", subcore_axis_name="s")


@functools.partial(
    pl.kernel,
    mesh=_sc_mesh,
    compiler_params=pltpu.CompilerParams(needs_layout_passes=False),
    out_type=jax.ShapeDtypeStruct((NTOK * D,), jnp.float32),
    scratch_types=[
        pltpu.VMEM((V * D + P * D,), jnp.float32),
        pltpu.VMEM((FW,), jnp.float32),
        pltpu.VMEM((TOK_W,), jnp.int32),
        pltpu.VMEM((CT * D,), jnp.float32),
        pltpu.VMEM((CT * D,), jnp.float32),
        pltpu.SemaphoreType.DMA,
        pltpu.SemaphoreType.DMA,
    ],
)
def _sc_embed(wp_hbm, x_hbm, out_hbm, wp_v, fused_v, x_v, buf0, buf1,
              sem0, sem1):
    wid = lax.axis_index("s") * 2 + lax.axis_index("c")
    base = pl.multiple_of(wid * TOK_W, TOK_W)
    pltpu.sync_copy(wp_hbm, wp_v)
    pltpu.sync_copy(x_hbm.at[pl.ds(base, TOK_W)], x_v)

    lane = lax.iota(jnp.int32, LANES)

    @plsc.parallel_loop(0, FW // LANES, 1, unroll=4)
    def build(i):
        p = i * LANES + lane
        r = p // DS
        c = jnp.minimum(p - r * DS, D - 1)
        l = r // V
        v = r - l * V
        wv = plsc.load_gather(wp_v, [v * D + c])
        pv = plsc.load_gather(wp_v, [V * D + l * D + c])
        fused_v[pl.ds(i * LANES, LANES)] = wv + pv

    half = lane // 8
    c1 = jnp.where(lane < 8, lane + 16, lane - 8)
    c2 = lane + 8

    bufs = (buf0, buf1)
    sems = (sem0, sem1)

    def compute_chunk(c, buf):
        @plsc.parallel_loop(0, GRP, 1, unroll=2)
        def group(g):
            t = pl.multiple_of(c * CT + g * LANES, LANES)
            xv = x_v[pl.ds(t, LANES)]
            lv = lax.rem(t + lane, P)
            row = (lv * V + xv) * DS
            ob = g * (LANES * D)
            for q in range(8):
                a = jnp.broadcast_to(row[2 * q], (LANES,))
                b = jnp.broadcast_to(row[2 * q + 1], (LANES,))
                ab = jnp.where(half == 0, a, b)
                o = ob + q * 48
                buf[pl.ds(o, LANES)] = plsc.load_gather(fused_v, [a + lane])
                buf[pl.ds(o + 16, LANES)] = plsc.load_gather(fused_v, [ab + c1])
                buf[pl.ds(o + 32, LANES)] = plsc.load_gather(fused_v, [b + c2])

    copies = []
    for c in range(NCH):
        bsel = c % 2
        if c >= 2:
            copies[c - 2].wait()
        compute_chunk(c, bufs[bsel])
        off = pl.multiple_of((base + c * CT) * D, CT * D)
        copies.append(
            pltpu.async_copy(bufs[bsel], out_hbm.at[pl.ds(off, CT * D)],
                             sems[bsel]))
    copies[-2].wait()
    copies[-1].wait()


def kernel(X, word_table, pos_table):
    wp = jnp.concatenate([word_table.reshape(V * D), pos_table.reshape(P * D)])
    x_flat = X.reshape(NTOK).astype(jnp.int32)
    out_flat = _sc_embed(wp, x_flat)
    return out_flat.reshape(B, P, D)


# speedup vs baseline: 2.4242x; 1.0837x over previous
import functools

import jax
import jax.numpy as jnp
from jax import lax
from jax.experimental import pallas as pl
from jax.experimental.pallas import tpu as pltpu
from jax.experimental.pallas import tpu_sc as plsc

B = 16384
P = 12
V = 28
D = 24
DS = 25
FW = P * V * DS
NTOK = B * P
NW = 32
TOK_W = NTOK // NW
CT = 1536
NCH = TOK_W // CT
GRP = CT // 16
LANES = 16

_sc_mesh = plsc.VectorSubcoreMesh(core_axis_name="c", subcore_axis_name="s")


@functools.partial(
    pl.kernel,
    mesh=_sc_mesh,
    compiler_params=pltpu.CompilerParams(needs_layout_passes=False),
    out_type=jax.ShapeDtypeStruct((NTOK * D // 128, 128), jnp.float32),
    scratch_types=[
        pltpu.VMEM((V * D + P * D,), jnp.float32),
        pltpu.VMEM((FW,), jnp.float32),
        pltpu.VMEM((TOK_W,), jnp.int32),
        pltpu.VMEM((CT * D // 128, 128), jnp.float32),
        pltpu.VMEM((CT * D // 128, 128), jnp.float32),
        pltpu.SemaphoreType.DMA,
        pltpu.SemaphoreType.DMA,
    ],
)
def _sc_embed(wp_hbm, x_hbm, out_hbm, wp_v, fused_v, x_v, buf0, buf1,
              sem0, sem1):
    wid = lax.axis_index("s") * 2 + lax.axis_index("c")
    base = pl.multiple_of(wid * TOK_W, TOK_W)
    pltpu.sync_copy(wp_hbm, wp_v)
    pltpu.sync_copy(x_hbm.at[pl.ds(base, TOK_W)], x_v)

    lane = lax.iota(jnp.int32, LANES)

    @plsc.parallel_loop(0, FW // LANES, 1, unroll=4)
    def build(i):
        p = i * LANES + lane
        r = p // DS
        c = jnp.minimum(p - r * DS, D - 1)
        l = r // V
        v = r - l * V
        wv = plsc.load_gather(wp_v, [v * D + c])
        pv = plsc.load_gather(wp_v, [V * D + l * D + c])
        fused_v[pl.ds(i * LANES, LANES)] = wv + pv

    half = lane // 8
    c1 = jnp.where(lane < 8, lane + 16, lane - 8)
    c2 = lane + 8

    bufs = (buf0, buf1)
    sems = (sem0, sem1)

    def compute_chunk(c, buf):
        @plsc.parallel_loop(0, GRP, 1, unroll=2)
        def group(g):
            t = pl.multiple_of(c * CT + g * LANES, LANES)
            xv = x_v[pl.ds(t, LANES)]
            lv = lax.rem(t + lane, P)
            row = (lv * V + xv) * DS
            gr = g * 3
            for q in range(8):
                a = jnp.broadcast_to(row[2 * q], (LANES,))
                b = jnp.broadcast_to(row[2 * q + 1], (LANES,))
                ab = jnp.where(half == 0, a, b)
                o = q * 48
                for m, idx in ((0, a + lane), (1, ab + c1), (2, b + c2)):
                    om = o + m * LANES
                    buf[gr + om // 128, pl.ds(om % 128, LANES)] = (
                        plsc.load_gather(fused_v, [idx]))

    copies = []
    for c in range(NCH):
        bsel = c % 2
        if c >= 2:
            copies[c - 2].wait()
        compute_chunk(c, bufs[bsel])
        off = pl.multiple_of((base + c * CT) * D // 128, CT * D // 128)
        copies.append(
            pltpu.async_copy(bufs[bsel], out_hbm.at[pl.ds(off, CT * D // 128)],
                             sems[bsel]))
    copies[-2].wait()
    copies[-1].wait()


def kernel(X, word_table, pos_table):
    wp = jnp.concatenate([word_table.reshape(V * D), pos_table.reshape(P * D)])
    x_flat = X.reshape(NTOK).astype(jnp.int32)
    out2 = _sc_embed(wp, x_flat)
    return out2.reshape(B, P, D)
